# initial kernel scaffold (unmeasured)
import jax
import jax.numpy as jnp
from jax import lax
from jax.experimental import pallas as pl
from jax.experimental.pallas import tpu as pltpu

T = 2048
N = 1024
C = 256
MAXC = T // C


def kernel(x, dest):
    my_x = lax.axis_index("x")

    order = jnp.argsort(dest, stable=True)
    xs = jnp.take(x, order, axis=0)

    k = jnp.sum((dest == my_x).astype(jnp.int32))
    m = T - k
    is0 = (my_x == 0).astype(jnp.int32)
    s = jnp.where(is0 == 1, k, 0)
    d = jnp.where(is0 == 1, 0, k)
    r = jnp.where(is0 == 1, k, 0)
    kk = jnp.where(is0 == 1, 0, m)
    nc = (m + C - 1) // C
    nck = (k + C - 1) // C
    scalars = jnp.stack([k, m, s, d, r, kk, nc, nck]).astype(jnp.int32)

    def body(sc_ref, x_ref, out_ref, send_sems, recv_sems):
        mx = lax.axis_index("x")
        my = lax.axis_index("y")
        peer = (1 - mx, my)

        m_ = sc_ref[1]
        s_ = sc_ref[2]
        d_ = sc_ref[3]
        r_ = sc_ref[4]
        kk_ = sc_ref[5]
        nc_ = sc_ref[6]
        nck_ = sc_ref[7]
        k_ = sc_ref[0]

        barrier_sem = pltpu.get_barrier_semaphore()
        pl.semaphore_signal(
            barrier_sem, inc=1,
            device_id=peer, device_id_type=pl.DeviceIdType.MESH,
        )
        pl.semaphore_wait(barrier_sem, 1)

        for i in range(MAXC):
            @pl.when(i < nc_)
            def _():
                off = jnp.minimum(i * C, m_ - C)
                rdma = pltpu.make_async_remote_copy(
                    src_ref=x_ref.at[pl.ds(s_ + off, C)],
                    dst_ref=out_ref.at[pl.ds(d_ + off, C)],
                    send_sem=send_sems.at[i],
                    recv_sem=recv_sems.at[i],
                    device_id=peer,
                    device_id_type=pl.DeviceIdType.MESH,
                )
                rdma.start()

        for i in range(MAXC):
            @pl.when(i < nck_)
            def _():
                off = jnp.minimum(i * C, k_ - C)
                out_ref[pl.ds(kk_ + off, C), :] = x_ref[pl.ds(kk_ + off, C), :]

        for i in range(MAXC):
            @pl.when(i < nc_)
            def _():
                off = jnp.minimum(i * C, m_ - C)
                recv = pltpu.make_async_remote_copy(
                    src_ref=x_ref.at[pl.ds(0, C)],
                    dst_ref=out_ref.at[pl.ds(r_ + off, C)],
                    send_sem=send_sems.at[i],
                    recv_sem=recv_sems.at[i],
                    device_id=peer,
                    device_id_type=pl.DeviceIdType.MESH,
                )
                recv.wait_recv()
                send = pltpu.make_async_remote_copy(
                    src_ref=x_ref.at[pl.ds(s_ + off, C)],
                    dst_ref=out_ref.at[pl.ds(0, C)],
                    send_sem=send_sems.at[i],
                    recv_sem=recv_sems.at[i],
                    device_id=peer,
                    device_id_type=pl.DeviceIdType.MESH,
                )
                send.wait_send()

    return pl.pallas_call(
        body,
        out_shape=jax.ShapeDtypeStruct((T, N), jnp.float32),
        in_specs=[
            pl.BlockSpec(memory_space=pltpu.SMEM),
            pl.BlockSpec(memory_space=pltpu.VMEM),
        ],
        out_specs=pl.BlockSpec(memory_space=pltpu.VMEM),
        scratch_shapes=[
            pltpu.SemaphoreType.DMA((MAXC,)),
            pltpu.SemaphoreType.DMA((MAXC,)),
        ],
        compiler_params=pltpu.CompilerParams(collective_id=0),
    )(scalars, xs)


# baseline (device time: 68773 ns/iter reference)
import jax
import jax.numpy as jnp
from jax import lax
from jax.experimental import pallas as pl
from jax.experimental.pallas import tpu as pltpu

T = 2048
N = 1024
C = 256
MAXC = T // C


def kernel(x, dest):
    my_x = lax.axis_index("x")

    order = jnp.argsort(dest, stable=True).astype(jnp.int32)

    k = jnp.sum((dest == my_x).astype(jnp.int32))
    m = T - k
    is0 = (my_x == 0).astype(jnp.int32)
    s = jnp.where(is0 == 1, k, 0)
    d = jnp.where(is0 == 1, 0, k)
    r = jnp.where(is0 == 1, k, 0)
    kk = jnp.where(is0 == 1, 0, m)
    nc = (m + C - 1) // C
    nck = (k + C - 1) // C
    scalars = jnp.stack([k, m, s, d, r, kk, nc, nck]).astype(jnp.int32)

    xr = x.reshape(T, 8, 128)

    def body(sc_ref, order_ref, x_ref, out_ref, stage_ref, send_sems, recv_sems):
        mx = lax.axis_index("x")
        my = lax.axis_index("y")
        peer = (1 - mx, my)

        k_ = sc_ref[0]
        m_ = sc_ref[1]
        s_ = sc_ref[2]
        d_ = sc_ref[3]
        r_ = sc_ref[4]
        kk_ = sc_ref[5]
        nc_ = sc_ref[6]
        nck_ = sc_ref[7]

        def copy_rows(src_base, dst_base, dst_ref):
            def body8(j8, _):
                base = j8 * 8
                for u in range(8):
                    j = base + u
                    idx = order_ref[src_base + j]
                    dst_ref[dst_base + j] = x_ref[idx]
                return 0
            lax.fori_loop(0, C // 8, body8, 0)

        barrier_sem = pltpu.get_barrier_semaphore()
        pl.semaphore_signal(
            barrier_sem, inc=1,
            device_id=peer, device_id_type=pl.DeviceIdType.MESH,
        )
        pl.semaphore_wait(barrier_sem, 1)

        for i in range(MAXC):
            @pl.when(i < nc_)
            def _():
                off = jnp.minimum(i * C, m_ - C)
                copy_rows(s_ + off, off, stage_ref)
                rdma = pltpu.make_async_remote_copy(
                    src_ref=stage_ref.at[pl.ds(off, C)],
                    dst_ref=out_ref.at[pl.ds(d_ + off, C)],
                    send_sem=send_sems.at[i],
                    recv_sem=recv_sems.at[i],
                    device_id=peer,
                    device_id_type=pl.DeviceIdType.MESH,
                )
                rdma.start()

        for i in range(MAXC):
            @pl.when(i < nck_)
            def _():
                off = jnp.minimum(i * C, k_ - C)
                copy_rows(kk_ + off, kk_ + off, out_ref)

        for i in range(MAXC):
            @pl.when(i < nc_)
            def _():
                off = jnp.minimum(i * C, m_ - C)
                recv = pltpu.make_async_remote_copy(
                    src_ref=stage_ref.at[pl.ds(0, C)],
                    dst_ref=out_ref.at[pl.ds(r_ + off, C)],
                    send_sem=send_sems.at[i],
                    recv_sem=recv_sems.at[i],
                    device_id=peer,
                    device_id_type=pl.DeviceIdType.MESH,
                )
                recv.wait_recv()
                send = pltpu.make_async_remote_copy(
                    src_ref=stage_ref.at[pl.ds(off, C)],
                    dst_ref=out_ref.at[pl.ds(0, C)],
                    send_sem=send_sems.at[i],
                    recv_sem=recv_sems.at[i],
                    device_id=peer,
                    device_id_type=pl.DeviceIdType.MESH,
                )
                send.wait_send()

    out = pl.pallas_call(
        body,
        out_shape=jax.ShapeDtypeStruct((T, 8, 128), jnp.float32),
        in_specs=[
            pl.BlockSpec(memory_space=pltpu.SMEM),
            pl.BlockSpec(memory_space=pltpu.SMEM),
            pl.BlockSpec(memory_space=pltpu.VMEM),
        ],
        out_specs=pl.BlockSpec(memory_space=pltpu.VMEM),
        scratch_shapes=[
            pltpu.VMEM((T, 8, 128), jnp.float32),
            pltpu.SemaphoreType.DMA((MAXC,)),
            pltpu.SemaphoreType.DMA((MAXC,)),
        ],
        compiler_params=pltpu.CompilerParams(collective_id=0),
    )(scalars, order, xr)
    return out.reshape(T, N)


# device time: 68099 ns/iter; 1.0099x vs baseline; 1.0099x over previous
import jax
import jax.numpy as jnp
from jax import lax
from jax.experimental import pallas as pl
from jax.experimental.pallas import tpu as pltpu

T = 2048
N = 1024
C = 256
MAXC = T // C


def kernel(x, dest):
    my_x = lax.axis_index("x")

    skey = jnp.sort(dest * 4096 + jnp.arange(T, dtype=jnp.int32))
    order = (skey & 4095).astype(jnp.int32)

    k = jnp.sum((dest == my_x).astype(jnp.int32))
    m = T - k
    is0 = (my_x == 0).astype(jnp.int32)
    s = jnp.where(is0 == 1, k, 0)
    d = jnp.where(is0 == 1, 0, k)
    r = jnp.where(is0 == 1, k, 0)
    kk = jnp.where(is0 == 1, 0, m)
    nc = (m + C - 1) // C
    nck = (k + C - 1) // C
    scalars = jnp.stack([k, m, s, d, r, kk, nc, nck]).astype(jnp.int32)

    xr = x.reshape(T, 8, 128)

    def body(sc_ref, order_ref, x_ref, out_ref, stage_ref, send_sems, recv_sems):
        mx = lax.axis_index("x")
        my = lax.axis_index("y")
        peer = (1 - mx, my)

        k_ = sc_ref[0]
        m_ = sc_ref[1]
        s_ = sc_ref[2]
        d_ = sc_ref[3]
        r_ = sc_ref[4]
        kk_ = sc_ref[5]
        nc_ = sc_ref[6]
        nck_ = sc_ref[7]

        def copy_rows(src_base, dst_base, dst_ref):
            def body8(j8, _):
                base = j8 * 8
                for u in range(8):
                    j = base + u
                    idx = order_ref[src_base + j]
                    dst_ref[dst_base + j] = x_ref[idx]
                return 0
            lax.fori_loop(0, C // 8, body8, 0)

        copy_rows(s_, 0, stage_ref)

        barrier_sem = pltpu.get_barrier_semaphore()
        pl.semaphore_signal(
            barrier_sem, inc=1,
            device_id=peer, device_id_type=pl.DeviceIdType.MESH,
        )
        pl.semaphore_wait(barrier_sem, 1)

        for i in range(MAXC):
            @pl.when(i < nc_)
            def _():
                off = jnp.minimum(i * C, m_ - C)
                if i > 0:
                    copy_rows(s_ + off, off, stage_ref)
                rdma = pltpu.make_async_remote_copy(
                    src_ref=stage_ref.at[pl.ds(off, C)],
                    dst_ref=out_ref.at[pl.ds(d_ + off, C)],
                    send_sem=send_sems.at[i],
                    recv_sem=recv_sems.at[i],
                    device_id=peer,
                    device_id_type=pl.DeviceIdType.MESH,
                )
                rdma.start()

        for i in range(MAXC):
            @pl.when(i < nck_)
            def _():
                off = jnp.minimum(i * C, k_ - C)
                copy_rows(kk_ + off, kk_ + off, out_ref)

        for i in range(MAXC):
            @pl.when(i < nc_)
            def _():
                off = jnp.minimum(i * C, m_ - C)
                recv = pltpu.make_async_remote_copy(
                    src_ref=stage_ref.at[pl.ds(0, C)],
                    dst_ref=out_ref.at[pl.ds(r_ + off, C)],
                    send_sem=send_sems.at[i],
                    recv_sem=recv_sems.at[i],
                    device_id=peer,
                    device_id_type=pl.DeviceIdType.MESH,
                )
                recv.wait_recv()
                send = pltpu.make_async_remote_copy(
                    src_ref=stage_ref.at[pl.ds(off, C)],
                    dst_ref=out_ref.at[pl.ds(0, C)],
                    send_sem=send_sems.at[i],
                    recv_sem=recv_sems.at[i],
                    device_id=peer,
                    device_id_type=pl.DeviceIdType.MESH,
                )
                send.wait_send()

    out = pl.pallas_call(
        body,
        out_shape=jax.ShapeDtypeStruct((T, 8, 128), jnp.float32),
        in_specs=[
            pl.BlockSpec(memory_space=pltpu.SMEM),
            pl.BlockSpec(memory_space=pltpu.SMEM),
            pl.BlockSpec(memory_space=pltpu.VMEM),
        ],
        out_specs=pl.BlockSpec(memory_space=pltpu.VMEM),
        scratch_shapes=[
            pltpu.VMEM((T, 8, 128), jnp.float32),
            pltpu.SemaphoreType.DMA((MAXC,)),
            pltpu.SemaphoreType.DMA((MAXC,)),
        ],
        compiler_params=pltpu.CompilerParams(collective_id=0),
    )(scalars, order, xr)
    return out.reshape(T, N)


# device time: 67354 ns/iter; 1.0211x vs baseline; 1.0111x over previous
import jax
import jax.numpy as jnp
from jax import lax
from jax.experimental import pallas as pl
from jax.experimental.pallas import tpu as pltpu

T = 2048
N = 1024
C = 256
MAXC = T // C


def kernel(x, dest):
    my_x = lax.axis_index("x")

    skey = jnp.sort(dest * 4096 + jnp.arange(T, dtype=jnp.int32), stable=False)
    order = (skey & 4095).astype(jnp.int32)

    k = jnp.sum((dest == my_x).astype(jnp.int32))
    m = T - k
    is0 = (my_x == 0).astype(jnp.int32)
    s = jnp.where(is0 == 1, k, 0)
    d = jnp.where(is0 == 1, 0, k)
    r = jnp.where(is0 == 1, k, 0)
    kk = jnp.where(is0 == 1, 0, m)
    nc = (m + C - 1) // C
    nck = (k + C - 1) // C
    scalars = jnp.stack([k, m, s, d, r, kk, nc, nck]).astype(jnp.int32)

    xr = x.reshape(T, 8, 128)

    def body(sc_ref, order_ref, x_ref, out_ref, stage_ref, send_sems, recv_sems):
        mx = lax.axis_index("x")
        my = lax.axis_index("y")
        peer = (1 - mx, my)

        k_ = sc_ref[0]
        m_ = sc_ref[1]
        s_ = sc_ref[2]
        d_ = sc_ref[3]
        r_ = sc_ref[4]
        kk_ = sc_ref[5]
        nc_ = sc_ref[6]
        nck_ = sc_ref[7]

        def copy_rows(src_base, dst_base, dst_ref):
            def body8(j8, _):
                base = j8 * 8
                for u in range(8):
                    j = base + u
                    idx = order_ref[src_base + j]
                    dst_ref[dst_base + j] = x_ref[idx]
                return 0
            lax.fori_loop(0, C // 8, body8, 0)

        copy_rows(s_, 0, stage_ref)

        barrier_sem = pltpu.get_barrier_semaphore()
        pl.semaphore_signal(
            barrier_sem, inc=1,
            device_id=peer, device_id_type=pl.DeviceIdType.MESH,
        )
        pl.semaphore_wait(barrier_sem, 1)

        for i in range(MAXC):
            @pl.when(i < nc_)
            def _():
                off = jnp.minimum(i * C, m_ - C)
                if i > 0:
                    copy_rows(s_ + off, off, stage_ref)
                rdma = pltpu.make_async_remote_copy(
                    src_ref=stage_ref.at[pl.ds(off, C)],
                    dst_ref=out_ref.at[pl.ds(d_ + off, C)],
                    send_sem=send_sems.at[i],
                    recv_sem=recv_sems.at[i],
                    device_id=peer,
                    device_id_type=pl.DeviceIdType.MESH,
                )
                rdma.start()

        for i in range(MAXC):
            @pl.when(i < nck_)
            def _():
                off = jnp.minimum(i * C, k_ - C)
                copy_rows(kk_ + off, kk_ + off, out_ref)

        for i in range(MAXC):
            @pl.when(i < nc_)
            def _():
                off = jnp.minimum(i * C, m_ - C)
                recv = pltpu.make_async_remote_copy(
                    src_ref=stage_ref.at[pl.ds(0, C)],
                    dst_ref=out_ref.at[pl.ds(r_ + off, C)],
                    send_sem=send_sems.at[i],
                    recv_sem=recv_sems.at[i],
                    device_id=peer,
                    device_id_type=pl.DeviceIdType.MESH,
                )
                recv.wait_recv()
                send = pltpu.make_async_remote_copy(
                    src_ref=stage_ref.at[pl.ds(off, C)],
                    dst_ref=out_ref.at[pl.ds(0, C)],
                    send_sem=send_sems.at[i],
                    recv_sem=recv_sems.at[i],
                    device_id=peer,
                    device_id_type=pl.DeviceIdType.MESH,
                )
                send.wait_send()

    out = pl.pallas_call(
        body,
        out_shape=jax.ShapeDtypeStruct((T, 8, 128), jnp.float32),
        in_specs=[
            pl.BlockSpec(memory_space=pltpu.SMEM),
            pl.BlockSpec(memory_space=pltpu.SMEM),
            pl.BlockSpec(memory_space=pltpu.VMEM),
        ],
        out_specs=pl.BlockSpec(memory_space=pltpu.VMEM),
        scratch_shapes=[
            pltpu.VMEM((T, 8, 128), jnp.float32),
            pltpu.SemaphoreType.DMA((MAXC,)),
            pltpu.SemaphoreType.DMA((MAXC,)),
        ],
        compiler_params=pltpu.CompilerParams(collective_id=0),
    )(scalars, order, xr)
    return out.reshape(T, N)


# device time: 67160 ns/iter; 1.0240x vs baseline; 1.0029x over previous
import jax
import jax.numpy as jnp
from jax import lax
from jax.experimental import pallas as pl
from jax.experimental.pallas import tpu as pltpu

T = 2048
N = 1024
C = 256
MAXC = T // C


def kernel(x, dest):
    my_x = lax.axis_index("x")

    skey = jnp.sort(dest * 4096 + jnp.arange(T, dtype=jnp.int32), stable=False)
    order = (skey & 4095).astype(jnp.int32)

    k = jnp.sum((dest == my_x).astype(jnp.int32)).reshape(1)

    xr = x.reshape(T, 8, 128)

    def body(sc_ref, order_ref, x_ref, out_ref, stage_ref, send_sems, recv_sems):
        mx = lax.axis_index("x")
        my = lax.axis_index("y")
        peer = (1 - mx, my)

        k_ = sc_ref[0]
        m_ = T - k_
        is0 = (mx == 0)
        s_ = jnp.where(is0, k_, 0)
        d_ = jnp.where(is0, 0, k_)
        r_ = jnp.where(is0, k_, 0)
        kk_ = jnp.where(is0, 0, m_)
        nc_ = (m_ + C - 1) // C
        nck_ = (k_ + C - 1) // C

        def copy_rows(src_base, dst_base, dst_ref):
            def body8(j8, _):
                base = j8 * 8
                for u in range(8):
                    j = base + u
                    idx = order_ref[src_base + j]
                    dst_ref[dst_base + j] = x_ref[idx]
                return 0
            lax.fori_loop(0, C // 8, body8, 0)

        copy_rows(s_, 0, stage_ref)

        barrier_sem = pltpu.get_barrier_semaphore()
        pl.semaphore_signal(
            barrier_sem, inc=1,
            device_id=peer, device_id_type=pl.DeviceIdType.MESH,
        )
        pl.semaphore_wait(barrier_sem, 1)

        for i in range(MAXC):
            @pl.when(i < nc_)
            def _():
                off = jnp.minimum(i * C, m_ - C)
                if i > 0:
                    copy_rows(s_ + off, off, stage_ref)
                rdma = pltpu.make_async_remote_copy(
                    src_ref=stage_ref.at[pl.ds(off, C)],
                    dst_ref=out_ref.at[pl.ds(d_ + off, C)],
                    send_sem=send_sems.at[i],
                    recv_sem=recv_sems.at[i],
                    device_id=peer,
                    device_id_type=pl.DeviceIdType.MESH,
                )
                rdma.start()

        for i in range(MAXC):
            @pl.when(i < nck_)
            def _():
                off = jnp.minimum(i * C, k_ - C)
                copy_rows(kk_ + off, kk_ + off, out_ref)

        for i in range(MAXC):
            @pl.when(i < nc_)
            def _():
                off = jnp.minimum(i * C, m_ - C)
                recv = pltpu.make_async_remote_copy(
                    src_ref=stage_ref.at[pl.ds(0, C)],
                    dst_ref=out_ref.at[pl.ds(r_ + off, C)],
                    send_sem=send_sems.at[i],
                    recv_sem=recv_sems.at[i],
                    device_id=peer,
                    device_id_type=pl.DeviceIdType.MESH,
                )
                recv.wait_recv()
                send = pltpu.make_async_remote_copy(
                    src_ref=stage_ref.at[pl.ds(off, C)],
                    dst_ref=out_ref.at[pl.ds(0, C)],
                    send_sem=send_sems.at[i],
                    recv_sem=recv_sems.at[i],
                    device_id=peer,
                    device_id_type=pl.DeviceIdType.MESH,
                )
                send.wait_send()

    out = pl.pallas_call(
        body,
        out_shape=jax.ShapeDtypeStruct((T, 8, 128), jnp.float32),
        in_specs=[
            pl.BlockSpec(memory_space=pltpu.SMEM),
            pl.BlockSpec(memory_space=pltpu.SMEM),
            pl.BlockSpec(memory_space=pltpu.VMEM),
        ],
        out_specs=pl.BlockSpec(memory_space=pltpu.VMEM),
        scratch_shapes=[
            pltpu.VMEM((T, 8, 128), jnp.float32),
            pltpu.SemaphoreType.DMA((MAXC,)),
            pltpu.SemaphoreType.DMA((MAXC,)),
        ],
        compiler_params=pltpu.CompilerParams(collective_id=0),
    )(k, order, xr)
    return out.reshape(T, N)
